# baseline (speedup 1.0000x reference)
import jax
import jax.numpy as jnp
from jax import lax
from jax.experimental import pallas as pl
from jax.experimental.pallas import tpu as pltpu
from jax.experimental.pallas import tpu_sc as plsc

_B, _L, _D, _V = 4096, 200, 64, 1000000
_N = _B * _L
_NC, _NS = 2, 16
_NW = _NC * _NS
_TOK_PER_W = _N // _NW
_C = 512
_KR = _C // 128
_CHUNKS = _TOK_PER_W // _C
_ROWS = _N // 128


def _sc_body(pos_ids_hbm, neg_ids_hbm, table_hbm, out_hbm,
             pos_rows, neg_rows, pos_idx, neg_idx, gsem, wsem):
    wid = lax.axis_index("s") * _NC + lax.axis_index("c")

    def chunk_body(ci, carry):
        base = wid * _TOK_PER_W + ci * _C
        row = wid * (_TOK_PER_W // 128) + ci * _KR

        pltpu.sync_copy(pos_ids_hbm.at[pl.ds(row, _KR), :], pos_idx)
        pltpu.sync_copy(neg_ids_hbm.at[pl.ds(row, _KR), :], neg_idx)

        copies = []
        for j in range(_KR):
            copies.append(pltpu.async_copy(
                table_hbm.at[pos_idx.at[j]],
                pos_rows.at[pl.ds(j * 128, 128), :], gsem))
            copies.append(pltpu.async_copy(
                table_hbm.at[neg_idx.at[j]],
                neg_rows.at[pl.ds(j * 128, 128), :], gsem))
        for cp in copies:
            cp.wait()

        wp = pltpu.async_copy(
            pos_rows, out_hbm.at[pl.ds(base, _C), pl.ds(0, _D)], wsem)
        wn = pltpu.async_copy(
            neg_rows, out_hbm.at[pl.ds(base, _C), pl.ds(_D, _D)], wsem)
        wp.wait()
        wn.wait()
        return carry

    lax.fori_loop(0, _CHUNKS, chunk_body, 0)


@jax.jit
def _sc_gather(pos2d, neg2d, table):
    mesh = plsc.VectorSubcoreMesh(core_axis_name="c", subcore_axis_name="s")
    k = pl.kernel(
        _sc_body,
        mesh=mesh,
        compiler_params=pltpu.CompilerParams(
            needs_layout_passes=False,
            use_tc_tiling_on_sc=False,
        ),
        out_type=jax.ShapeDtypeStruct((_N, 2 * _D), jnp.float32),
        scratch_types=[
            pltpu.VMEM((_C, _D), jnp.float32),
            pltpu.VMEM((_C, _D), jnp.float32),
            pltpu.VMEM((_KR, 128), jnp.int32),
            pltpu.VMEM((_KR, 128), jnp.int32),
            pltpu.SemaphoreType.DMA,
            pltpu.SemaphoreType.DMA,
        ],
    )
    return k(pos2d, neg2d, table)


_GL = 25
_GB = 8
_LB = _L // _GL
_BB = _B // _GB


def _tc_body(seq_ref, comb_ref, ids_ref, out_ref, acc_ref, nv_ref):
    i = pl.program_id(0)
    j = pl.program_id(1)

    @pl.when((i == 0) & (j == 0))
    def _init():
        acc_ref[0] = 0.0
        nv_ref[0] = 0.0

    s = seq_ref[...]
    ct = jnp.swapaxes(comb_ref[...], 1, 2)
    p = ct[:, 0:_D, :]
    n = ct[:, _D:2 * _D, :]
    dp = jnp.sum(s * p, axis=1)
    dn = jnp.sum(s * n, axis=1)
    m = (ids_ref[...] != 0).astype(jnp.float32)
    sp = jnp.log1p(jnp.exp(-jnp.abs(dp))) + jnp.maximum(-dp, 0.0)
    sn = jnp.log1p(jnp.exp(-jnp.abs(dn))) + jnp.maximum(dn, 0.0)
    acc_ref[0] += ((sp + sn) * m).sum()
    nv_ref[0] += m.sum()

    @pl.when((i == _GL - 1) & (j == _GB - 1))
    def _fin():
        out_ref[0, 0] = acc_ref[0] / (2.0 * nv_ref[0])


@jax.jit
def _tc_loss(seq_t, comb3, ids_lm):
    f = pl.pallas_call(
        _tc_body,
        grid=(_GL, _GB),
        in_specs=[
            pl.BlockSpec((_LB, _D, _BB), lambda i, j: (i, 0, j)),
            pl.BlockSpec((_LB, _BB, 2 * _D), lambda i, j: (i, j, 0)),
            pl.BlockSpec((_LB, _BB), lambda i, j: (i, j)),
        ],
        out_specs=pl.BlockSpec(memory_space=pltpu.SMEM),
        out_shape=jax.ShapeDtypeStruct((1, 1), jnp.float32),
        scratch_shapes=[
            pltpu.SMEM((1,), jnp.float32),
            pltpu.SMEM((1,), jnp.float32),
        ],
    )
    return f(seq_t, comb3, ids_lm)[0, 0]


def kernel(sequence_output, positive_target_ids, negative_target_ids, item_embedding_table):
    seq_t = jnp.transpose(sequence_output, (1, 2, 0))
    pos_lm = jnp.transpose(positive_target_ids, (1, 0))
    neg_lm = jnp.transpose(negative_target_ids, (1, 0))
    pos2d = pos_lm.reshape(_N // 128, 128)
    neg2d = neg_lm.reshape(_N // 128, 128)
    comb = _sc_gather(pos2d, neg2d, item_embedding_table)
    comb3 = comb.reshape(_L, _B, 2 * _D)
    return _tc_loss(seq_t, comb3, pos_lm)


# speedup vs baseline: 1.0963x; 1.0963x over previous
import jax
import jax.numpy as jnp
from jax import lax
from jax.experimental import pallas as pl
from jax.experimental.pallas import tpu as pltpu
from jax.experimental.pallas import tpu_sc as plsc

_B, _L, _D, _V = 4096, 200, 64, 1000000
_N = _B * _L
_NC, _NS = 2, 16
_NW = _NC * _NS
_C = 512
_KR = _C // 128
_L1, _L2 = 96, 104
_LB = 4


def _make_sc_body(nt):
    tok_per_w = nt // _NW
    chunks = tok_per_w // _C

    def _sc_body(pos_ids_hbm, neg_ids_hbm, table_hbm, out_hbm,
                 pos_rows, neg_rows, pos_idx, neg_idx, gsem, wsem):
        wid = lax.axis_index("s") * _NC + lax.axis_index("c")

        def chunk_body(ci, carry):
            base = wid * tok_per_w + ci * _C
            row = wid * (tok_per_w // 128) + ci * _KR

            pltpu.sync_copy(pos_ids_hbm.at[pl.ds(row, _KR), :], pos_idx)
            pltpu.sync_copy(neg_ids_hbm.at[pl.ds(row, _KR), :], neg_idx)

            copies = []
            for j in range(_KR):
                copies.append(pltpu.async_copy(
                    table_hbm.at[pos_idx.at[j]],
                    pos_rows.at[pl.ds(j * 128, 128), :], gsem))
                copies.append(pltpu.async_copy(
                    table_hbm.at[neg_idx.at[j]],
                    neg_rows.at[pl.ds(j * 128, 128), :], gsem))
            for cp in copies:
                cp.wait()

            wp = pltpu.async_copy(
                pos_rows, out_hbm.at[pl.ds(base, _C), pl.ds(0, _D)], wsem)
            wn = pltpu.async_copy(
                neg_rows, out_hbm.at[pl.ds(base, _C), pl.ds(_D, _D)], wsem)
            wp.wait()
            wn.wait()
            return carry

        lax.fori_loop(0, chunks, chunk_body, 0)

    return _sc_body


def _sc_gather(pos2d, neg2d, table, nt):
    mesh = plsc.VectorSubcoreMesh(core_axis_name="c", subcore_axis_name="s")
    k = pl.kernel(
        _make_sc_body(nt),
        mesh=mesh,
        compiler_params=pltpu.CompilerParams(
            needs_layout_passes=False,
            use_tc_tiling_on_sc=False,
        ),
        out_type=jax.ShapeDtypeStruct((nt, 2 * _D), jnp.float32),
        scratch_types=[
            pltpu.VMEM((_C, _D), jnp.float32),
            pltpu.VMEM((_C, _D), jnp.float32),
            pltpu.VMEM((_KR, 128), jnp.int32),
            pltpu.VMEM((_KR, 128), jnp.int32),
            pltpu.SemaphoreType.DMA,
            pltpu.SemaphoreType.DMA,
        ],
    )
    return k(pos2d, neg2d, table)


def _tc_body_partial(seq_ref, comb_ref, ids_ref, out_ref, acc_ref, nv_ref):
    i = pl.program_id(0)

    @pl.when(i == 0)
    def _init():
        acc_ref[0] = 0.0
        nv_ref[0] = 0.0

    s = seq_ref[...]
    ct = jnp.swapaxes(comb_ref[...], 1, 2)
    p = ct[:, 0:_D, :]
    n = ct[:, _D:2 * _D, :]
    dp = jnp.sum(s * p, axis=1)
    dn = jnp.sum(s * n, axis=1)
    m = (ids_ref[...].reshape(_LB, _B) != 0).astype(jnp.float32)
    sp = jnp.log1p(jnp.exp(-jnp.abs(dp))) + jnp.maximum(-dp, 0.0)
    sn = jnp.log1p(jnp.exp(-jnp.abs(dn))) + jnp.maximum(dn, 0.0)
    acc_ref[0] += ((sp + sn) * m).sum()
    nv_ref[0] += m.sum()

    ng = pl.num_programs(0)

    @pl.when(i == ng - 1)
    def _fin():
        out_ref[0] = acc_ref[0]
        out_ref[1] = nv_ref[0]


def _tc_body_final(seq_ref, comb_ref, ids_ref, part_ref,
                   out_ref, acc_ref, nv_ref):
    i = pl.program_id(0)

    @pl.when(i == 0)
    def _init():
        acc_ref[0] = part_ref[0]
        nv_ref[0] = part_ref[1]

    s = seq_ref[...]
    ct = jnp.swapaxes(comb_ref[...], 1, 2)
    p = ct[:, 0:_D, :]
    n = ct[:, _D:2 * _D, :]
    dp = jnp.sum(s * p, axis=1)
    dn = jnp.sum(s * n, axis=1)
    m = (ids_ref[...].reshape(_LB, _B) != 0).astype(jnp.float32)
    sp = jnp.log1p(jnp.exp(-jnp.abs(dp))) + jnp.maximum(-dp, 0.0)
    sn = jnp.log1p(jnp.exp(-jnp.abs(dn))) + jnp.maximum(dn, 0.0)
    acc_ref[0] += ((sp + sn) * m).sum()
    nv_ref[0] += m.sum()

    ng = pl.num_programs(0)

    @pl.when(i == ng - 1)
    def _fin():
        out_ref[0, 0] = acc_ref[0] / (2.0 * nv_ref[0])


def _tc_partial(seq_t, comb3, ids3, l0, nl):
    g = nl // _LB
    g0 = l0 // _LB
    f = pl.pallas_call(
        _tc_body_partial,
        grid=(g,),
        in_specs=[
            pl.BlockSpec((_LB, _D, _B), lambda i: (i + g0, 0, 0)),
            pl.BlockSpec((_LB, _B, 2 * _D), lambda i: (i, 0, 0)),
            pl.BlockSpec((1, _LB, _B), lambda i: (i + g0, 0, 0)),
        ],
        out_specs=pl.BlockSpec(memory_space=pltpu.SMEM),
        out_shape=jax.ShapeDtypeStruct((2,), jnp.float32),
        scratch_shapes=[
            pltpu.SMEM((1,), jnp.float32),
            pltpu.SMEM((1,), jnp.float32),
        ],
    )
    return f(seq_t, comb3, ids3)


def _tc_final(seq_t, comb3, ids3, part, l0, nl):
    g = nl // _LB
    g0 = l0 // _LB
    f = pl.pallas_call(
        _tc_body_final,
        grid=(g,),
        in_specs=[
            pl.BlockSpec((_LB, _D, _B), lambda i: (i + g0, 0, 0)),
            pl.BlockSpec((_LB, _B, 2 * _D), lambda i: (i, 0, 0)),
            pl.BlockSpec((1, _LB, _B), lambda i: (i + g0, 0, 0)),
            pl.BlockSpec(memory_space=pltpu.SMEM),
        ],
        out_specs=pl.BlockSpec(memory_space=pltpu.SMEM),
        out_shape=jax.ShapeDtypeStruct((1, 1), jnp.float32),
        scratch_shapes=[
            pltpu.SMEM((1,), jnp.float32),
            pltpu.SMEM((1,), jnp.float32),
        ],
    )
    return f(seq_t, comb3, ids3, part)[0, 0]


@jax.jit
def _run(sequence_output, positive_target_ids, negative_target_ids, table):
    seq_t = jnp.transpose(sequence_output, (1, 2, 0))
    pos_lm = jnp.transpose(positive_target_ids, (1, 0))
    neg_lm = jnp.transpose(negative_target_ids, (1, 0))
    ids3 = pos_lm.reshape(_L // _LB, _LB, _B)

    n1 = _L1 * _B
    n2 = _L2 * _B
    pos1 = pos_lm[:_L1].reshape(n1 // 128, 128)
    neg1 = neg_lm[:_L1].reshape(n1 // 128, 128)
    pos2 = pos_lm[_L1:].reshape(n2 // 128, 128)
    neg2 = neg_lm[_L1:].reshape(n2 // 128, 128)

    comb1 = _sc_gather(pos1, neg1, table, n1).reshape(_L1, _B, 2 * _D)
    comb2 = _sc_gather(pos2, neg2, table, n2).reshape(_L2, _B, 2 * _D)

    part = _tc_partial(seq_t, comb1, ids3, 0, _L1)
    return _tc_final(seq_t, comb2, ids3, part, _L1, _L2)


def kernel(sequence_output, positive_target_ids, negative_target_ids, item_embedding_table):
    return _run(sequence_output, positive_target_ids,
                negative_target_ids, item_embedding_table)


# speedup vs baseline: 1.1085x; 1.0111x over previous
import jax
import jax.numpy as jnp
from jax import lax
from jax.experimental import pallas as pl
from jax.experimental.pallas import tpu as pltpu
from jax.experimental.pallas import tpu_sc as plsc

_B, _L, _D, _V = 4096, 200, 64, 1000000
_N = _B * _L
_NC, _NS = 2, 16
_NW = _NC * _NS
_C = 256
_KR = _C // 128
_L1, _L2 = 96, 104
_LB = 4


def _make_sc_body(nt):
    tok_per_w = nt // _NW
    chunks = tok_per_w // _C
    pairs = chunks // 2

    def _sc_body(pos_ids_hbm, neg_ids_hbm, table_hbm, out_hbm,
                 pr0, nr0, pr1, nr1, pi0, ni0, pi1, ni1,
                 gs0, gs1, ws0, ws1):
        wid = lax.axis_index("s") * _NC + lax.axis_index("c")
        base0 = wid * tok_per_w
        row0 = wid * (tok_per_w // 128)

        def idx_load(ci, pi, ni):
            row = row0 + ci * _KR
            pltpu.sync_copy(pos_ids_hbm.at[pl.ds(row, _KR), :], pi)
            pltpu.sync_copy(neg_ids_hbm.at[pl.ds(row, _KR), :], ni)

        def fire_gath(pi, ni, pr, nr, gs):
            for j in range(_KR):
                pltpu.async_copy(table_hbm.at[pi.at[j]],
                                 pr.at[pl.ds(j * 128, 128), :], gs)
                pltpu.async_copy(table_hbm.at[ni.at[j]],
                                 nr.at[pl.ds(j * 128, 128), :], gs)

        def wait_gath(pr, nr, gs):
            pltpu.make_async_copy(table_hbm.at[pl.ds(0, _C), :], pr, gs).wait()
            pltpu.make_async_copy(table_hbm.at[pl.ds(0, _C), :], nr, gs).wait()

        def fire_write(ci, pr, nr, ws):
            base = base0 + ci * _C
            pltpu.async_copy(pr, out_hbm.at[pl.ds(base, _C), pl.ds(0, _D)], ws)
            pltpu.async_copy(nr, out_hbm.at[pl.ds(base, _C), pl.ds(_D, _D)], ws)

        def wait_write(pr, nr, ws):
            pltpu.make_async_copy(
                pr, out_hbm.at[pl.ds(0, _C), pl.ds(0, _D)], ws).wait()
            pltpu.make_async_copy(
                nr, out_hbm.at[pl.ds(0, _C), pl.ds(_D, _D)], ws).wait()

        idx_load(0, pi0, ni0)
        fire_gath(pi0, ni0, pr0, nr0, gs0)

        def pair_body(cp, carry):
            e = cp * 2
            o = e + 1
            idx_load(o, pi1, ni1)
            fire_gath(pi1, ni1, pr1, nr1, gs1)
            wait_gath(pr0, nr0, gs0)
            fire_write(e, pr0, nr0, ws0)
            wait_write(pr0, nr0, ws0)

            @pl.when(cp < pairs - 1)
            def _prefetch():
                idx_load(e + 2, pi0, ni0)
                fire_gath(pi0, ni0, pr0, nr0, gs0)

            wait_gath(pr1, nr1, gs1)
            fire_write(o, pr1, nr1, ws1)
            wait_write(pr1, nr1, ws1)
            return carry

        lax.fori_loop(0, pairs, pair_body, 0)

    return _sc_body


def _sc_gather(pos2d, neg2d, table, nt):
    mesh = plsc.VectorSubcoreMesh(core_axis_name="c", subcore_axis_name="s")
    k = pl.kernel(
        _make_sc_body(nt),
        mesh=mesh,
        compiler_params=pltpu.CompilerParams(
            needs_layout_passes=False,
            use_tc_tiling_on_sc=False,
        ),
        out_type=jax.ShapeDtypeStruct((nt, 2 * _D), jnp.float32),
        scratch_types=[
            pltpu.VMEM((_C, _D), jnp.float32),
            pltpu.VMEM((_C, _D), jnp.float32),
            pltpu.VMEM((_C, _D), jnp.float32),
            pltpu.VMEM((_C, _D), jnp.float32),
            pltpu.VMEM((_KR, 128), jnp.int32),
            pltpu.VMEM((_KR, 128), jnp.int32),
            pltpu.VMEM((_KR, 128), jnp.int32),
            pltpu.VMEM((_KR, 128), jnp.int32),
            pltpu.SemaphoreType.DMA,
            pltpu.SemaphoreType.DMA,
            pltpu.SemaphoreType.DMA,
            pltpu.SemaphoreType.DMA,
        ],
    )
    return k(pos2d, neg2d, table)


def _tc_body_partial(seq_ref, comb_ref, ids_ref, out_ref, acc_ref, nv_ref):
    i = pl.program_id(0)

    @pl.when(i == 0)
    def _init():
        acc_ref[0] = 0.0
        nv_ref[0] = 0.0

    s = seq_ref[...]
    ct = jnp.swapaxes(comb_ref[...], 1, 2)
    p = ct[:, 0:_D, :]
    n = ct[:, _D:2 * _D, :]
    dp = jnp.sum(s * p, axis=1)
    dn = jnp.sum(s * n, axis=1)
    m = (ids_ref[...].reshape(_LB, _B) != 0).astype(jnp.float32)
    sp = jnp.log1p(jnp.exp(-jnp.abs(dp))) + jnp.maximum(-dp, 0.0)
    sn = jnp.log1p(jnp.exp(-jnp.abs(dn))) + jnp.maximum(dn, 0.0)
    acc_ref[0] += ((sp + sn) * m).sum()
    nv_ref[0] += m.sum()

    ng = pl.num_programs(0)

    @pl.when(i == ng - 1)
    def _fin():
        out_ref[0] = acc_ref[0]
        out_ref[1] = nv_ref[0]


def _tc_body_final(seq_ref, comb_ref, ids_ref, part_ref,
                   out_ref, acc_ref, nv_ref):
    i = pl.program_id(0)

    @pl.when(i == 0)
    def _init():
        acc_ref[0] = part_ref[0]
        nv_ref[0] = part_ref[1]

    s = seq_ref[...]
    ct = jnp.swapaxes(comb_ref[...], 1, 2)
    p = ct[:, 0:_D, :]
    n = ct[:, _D:2 * _D, :]
    dp = jnp.sum(s * p, axis=1)
    dn = jnp.sum(s * n, axis=1)
    m = (ids_ref[...].reshape(_LB, _B) != 0).astype(jnp.float32)
    sp = jnp.log1p(jnp.exp(-jnp.abs(dp))) + jnp.maximum(-dp, 0.0)
    sn = jnp.log1p(jnp.exp(-jnp.abs(dn))) + jnp.maximum(dn, 0.0)
    acc_ref[0] += ((sp + sn) * m).sum()
    nv_ref[0] += m.sum()

    ng = pl.num_programs(0)

    @pl.when(i == ng - 1)
    def _fin():
        out_ref[0, 0] = acc_ref[0] / (2.0 * nv_ref[0])


def _tc_partial(seq_t, comb3, ids3, l0, nl):
    g = nl // _LB
    g0 = l0 // _LB
    f = pl.pallas_call(
        _tc_body_partial,
        grid=(g,),
        in_specs=[
            pl.BlockSpec((_LB, _D, _B), lambda i: (i + g0, 0, 0)),
            pl.BlockSpec((_LB, _B, 2 * _D), lambda i: (i, 0, 0)),
            pl.BlockSpec((1, _LB, _B), lambda i: (i + g0, 0, 0)),
        ],
        out_specs=pl.BlockSpec(memory_space=pltpu.SMEM),
        out_shape=jax.ShapeDtypeStruct((2,), jnp.float32),
        scratch_shapes=[
            pltpu.SMEM((1,), jnp.float32),
            pltpu.SMEM((1,), jnp.float32),
        ],
    )
    return f(seq_t, comb3, ids3)


def _tc_final(seq_t, comb3, ids3, part, l0, nl):
    g = nl // _LB
    g0 = l0 // _LB
    f = pl.pallas_call(
        _tc_body_final,
        grid=(g,),
        in_specs=[
            pl.BlockSpec((_LB, _D, _B), lambda i: (i + g0, 0, 0)),
            pl.BlockSpec((_LB, _B, 2 * _D), lambda i: (i, 0, 0)),
            pl.BlockSpec((1, _LB, _B), lambda i: (i + g0, 0, 0)),
            pl.BlockSpec(memory_space=pltpu.SMEM),
        ],
        out_specs=pl.BlockSpec(memory_space=pltpu.SMEM),
        out_shape=jax.ShapeDtypeStruct((1, 1), jnp.float32),
        scratch_shapes=[
            pltpu.SMEM((1,), jnp.float32),
            pltpu.SMEM((1,), jnp.float32),
        ],
    )
    return f(seq_t, comb3, ids3, part)[0, 0]


@jax.jit
def _run(sequence_output, positive_target_ids, negative_target_ids, table):
    seq_t = jnp.transpose(sequence_output, (1, 2, 0))
    pos_lm = jnp.transpose(positive_target_ids, (1, 0))
    neg_lm = jnp.transpose(negative_target_ids, (1, 0))
    ids3 = pos_lm.reshape(_L // _LB, _LB, _B)

    n1 = _L1 * _B
    n2 = _L2 * _B
    pos1 = pos_lm[:_L1].reshape(n1 // 128, 128)
    neg1 = neg_lm[:_L1].reshape(n1 // 128, 128)
    pos2 = pos_lm[_L1:].reshape(n2 // 128, 128)
    neg2 = neg_lm[_L1:].reshape(n2 // 128, 128)

    comb1 = _sc_gather(pos1, neg1, table, n1).reshape(_L1, _B, 2 * _D)
    comb2 = _sc_gather(pos2, neg2, table, n2).reshape(_L2, _B, 2 * _D)

    part = _tc_partial(seq_t, comb1, ids3, 0, _L1)
    return _tc_final(seq_t, comb2, ids3, part, _L1, _L2)


def kernel(sequence_output, positive_target_ids, negative_target_ids, item_embedding_table):
    return _run(sequence_output, positive_target_ids,
                negative_target_ids, item_embedding_table)


# speedup vs baseline: 1.1121x; 1.0033x over previous
import jax
import jax.numpy as jnp
from jax import lax
from jax.experimental import pallas as pl
from jax.experimental.pallas import tpu as pltpu
from jax.experimental.pallas import tpu_sc as plsc

_B, _L, _D, _V = 4096, 200, 64, 1000000
_N = _B * _L
_NC, _NS = 2, 16
_NW = _NC * _NS
_C = 256
_KR = _C // 128
_L1, _L2 = 96, 104
_LB = 4


def _make_sc_body(nt):
    tok_per_w = nt // _NW
    chunks = tok_per_w // _C
    pairs = chunks // 2

    def _sc_body(pos_ids_hbm, neg_ids_hbm, table_hbm, out_hbm,
                 pr0, nr0, pr1, nr1, pi0, ni0, pi1, ni1,
                 gs0, gs1, ws0, ws1):
        wid = lax.axis_index("s") * _NC + lax.axis_index("c")
        base0 = wid * tok_per_w
        row0 = wid * (tok_per_w // 128)

        def idx_load(ci, pi, ni):
            row = row0 + ci * _KR
            pltpu.sync_copy(pos_ids_hbm.at[pl.ds(row, _KR), :], pi)
            pltpu.sync_copy(neg_ids_hbm.at[pl.ds(row, _KR), :], ni)

        def fire_gath(pi, ni, pr, nr, gs):
            for j in range(_KR):
                pltpu.async_copy(table_hbm.at[pi.at[j]],
                                 pr.at[pl.ds(j * 128, 128), :], gs)
                pltpu.async_copy(table_hbm.at[ni.at[j]],
                                 nr.at[pl.ds(j * 128, 128), :], gs)

        def wait_gath(pr, nr, gs):
            pltpu.make_async_copy(table_hbm.at[pl.ds(0, _C), :], pr, gs).wait()
            pltpu.make_async_copy(table_hbm.at[pl.ds(0, _C), :], nr, gs).wait()

        def fire_write(ci, pr, nr, ws):
            base = base0 + ci * _C
            pltpu.async_copy(pr, out_hbm.at[pl.ds(base, _C), pl.ds(0, _D)], ws)
            pltpu.async_copy(nr, out_hbm.at[pl.ds(base, _C), pl.ds(_D, _D)], ws)

        def wait_write(pr, nr, ws):
            pltpu.make_async_copy(
                pr, out_hbm.at[pl.ds(0, _C), pl.ds(0, _D)], ws).wait()
            pltpu.make_async_copy(
                nr, out_hbm.at[pl.ds(0, _C), pl.ds(_D, _D)], ws).wait()

        idx_load(0, pi0, ni0)
        fire_gath(pi0, ni0, pr0, nr0, gs0)

        def pair_body(cp, carry):
            e = cp * 2
            o = e + 1
            idx_load(o, pi1, ni1)
            fire_gath(pi1, ni1, pr1, nr1, gs1)
            wait_gath(pr0, nr0, gs0)
            fire_write(e, pr0, nr0, ws0)
            wait_write(pr0, nr0, ws0)

            @pl.when(cp < pairs - 1)
            def _prefetch():
                idx_load(e + 2, pi0, ni0)
                fire_gath(pi0, ni0, pr0, nr0, gs0)

            wait_gath(pr1, nr1, gs1)
            fire_write(o, pr1, nr1, ws1)
            wait_write(pr1, nr1, ws1)
            return carry

        lax.fori_loop(0, pairs, pair_body, 0)

    return _sc_body


def _sc_gather(pos2d, neg2d, table, nt):
    mesh = plsc.VectorSubcoreMesh(core_axis_name="c", subcore_axis_name="s")
    k = pl.kernel(
        _make_sc_body(nt),
        mesh=mesh,
        compiler_params=pltpu.CompilerParams(
            needs_layout_passes=False,
            use_tc_tiling_on_sc=False,
        ),
        out_type=jax.ShapeDtypeStruct((nt, 2 * _D), jnp.float32),
        scratch_types=[
            pltpu.VMEM((_C, _D), jnp.float32),
            pltpu.VMEM((_C, _D), jnp.float32),
            pltpu.VMEM((_C, _D), jnp.float32),
            pltpu.VMEM((_C, _D), jnp.float32),
            pltpu.VMEM((_KR, 128), jnp.int32),
            pltpu.VMEM((_KR, 128), jnp.int32),
            pltpu.VMEM((_KR, 128), jnp.int32),
            pltpu.VMEM((_KR, 128), jnp.int32),
            pltpu.SemaphoreType.DMA,
            pltpu.SemaphoreType.DMA,
            pltpu.SemaphoreType.DMA,
            pltpu.SemaphoreType.DMA,
        ],
    )
    return k(pos2d, neg2d, table)


def _make_tc_body(final):
    def body(seq_ref, comb_ref, ids_ref, part_ref, out_ref, acc_ref, nv_ref):
        i = pl.program_id(0)

        @pl.when(i == 0)
        def _init():
            acc_ref[0] = part_ref[0]
            nv_ref[0] = part_ref[1]

        s = seq_ref[...]
        ct = jnp.swapaxes(comb_ref[...], 1, 2)
        p = ct[:, 0:_D, :]
        n = ct[:, _D:2 * _D, :]
        dp = jnp.sum(s * p, axis=1)
        dn = jnp.sum(s * n, axis=1)
        m = (ids_ref[...].reshape(_LB, _B) != 0).astype(jnp.float32)
        sp = jnp.log1p(jnp.exp(-jnp.abs(dp))) + jnp.maximum(-dp, 0.0)
        sn = jnp.log1p(jnp.exp(-jnp.abs(dn))) + jnp.maximum(dn, 0.0)
        acc_ref[0] += ((sp + sn) * m).sum()
        nv_ref[0] += m.sum()

        ng = pl.num_programs(0)

        @pl.when(i == ng - 1)
        def _fin():
            if final:
                out_ref[0] = acc_ref[0] / (2.0 * nv_ref[0])
                out_ref[1] = 0.0
            else:
                out_ref[0] = acc_ref[0]
                out_ref[1] = nv_ref[0]

    return body


def _tc_phase(seq_t, comb3, ids3, part, l0, nl, final):
    g = nl // _LB
    g0 = l0 // _LB
    f = pl.pallas_call(
        _make_tc_body(final),
        grid=(g,),
        in_specs=[
            pl.BlockSpec((_LB, _D, _B), lambda i: (i + g0, 0, 0)),
            pl.BlockSpec((_LB, _B, 2 * _D), lambda i: (i, 0, 0)),
            pl.BlockSpec((1, _LB, _B), lambda i: (i + g0, 0, 0)),
            pl.BlockSpec(memory_space=pltpu.SMEM),
        ],
        out_specs=pl.BlockSpec(memory_space=pltpu.SMEM),
        out_shape=jax.ShapeDtypeStruct((2,), jnp.float32),
        scratch_shapes=[
            pltpu.SMEM((1,), jnp.float32),
            pltpu.SMEM((1,), jnp.float32),
        ],
    )
    return f(seq_t, comb3, ids3, part)


_PHASES = (48, 48, 48, 56)


@jax.jit
def _run(sequence_output, positive_target_ids, negative_target_ids, table):
    seq_t = jnp.transpose(sequence_output, (1, 2, 0))
    pos_lm = jnp.transpose(positive_target_ids, (1, 0))
    neg_lm = jnp.transpose(negative_target_ids, (1, 0))
    ids3 = pos_lm.reshape(_L // _LB, _LB, _B)

    combs = []
    l0 = 0
    for nl in _PHASES:
        nt = nl * _B
        posq = pos_lm[l0:l0 + nl].reshape(nt // 128, 128)
        negq = neg_lm[l0:l0 + nl].reshape(nt // 128, 128)
        combs.append(_sc_gather(posq, negq, table, nt).reshape(nl, _B, 2 * _D))
        l0 += nl

    part = jnp.zeros((2,), jnp.float32)
    l0 = 0
    for k, nl in enumerate(_PHASES):
        part = _tc_phase(seq_t, combs[k], ids3, part, l0, nl,
                         final=(k == len(_PHASES) - 1))
        l0 += nl
    return part[0]


def kernel(sequence_output, positive_target_ids, negative_target_ids, item_embedding_table):
    return _run(sequence_output, positive_target_ids,
                negative_target_ids, item_embedding_table)
